# baseline (speedup 1.0000x reference)
import functools

import jax
import jax.numpy as jnp
from jax import lax
from jax.experimental import pallas as pl
from jax.experimental.pallas import tpu as pltpu
from jax.experimental.pallas import tpu_sc as plsc

B, M, N = 8, 1024, 256
H, W = 128, 128
HW = H * W
NCH = 4
CHUNKS_PER_WORKER = 16


def _sc_body(xt_hbm, locy_hbm, locx_hbm, out_hbm, yv, xv, idxv, xs, buf):
    c = lax.axis_index("c")
    s = lax.axis_index("s")
    wid = c * 16 + s
    b = wid // 4
    band = wid % 4

    pltpu.sync_copy(locy_hbm.at[b], yv)
    pltpu.sync_copy(locx_hbm.at[b], xv)

    def idx_body(g, carry):
        ys = yv[pl.ds(g * 16, 16)]
        xcs = xv[pl.ds(g * 16, 16)]
        idxv[pl.ds(g * 16, 16)] = ys * W + xcs
        return carry

    lax.fori_loop(0, M // 16, idx_body, 0)

    lanes = lax.iota(jnp.int32, 16)
    m4 = lanes < NCH
    gat_base = jnp.where(m4, lanes * M, 0)
    sct_base = jnp.where(m4, lanes * HW, 0)
    zeros16 = jnp.zeros((16,), jnp.float32)

    def chunk_body(t, carry):
        cg = band * CHUNKS_PER_WORKER + t

        pltpu.sync_copy(xt_hbm.at[b, pl.ds(cg * NCH * M, NCH * M)], xs)

        def zero_body(i, zc):
            buf[pl.ds(i * 16, 16)] = zeros16
            return zc

        lax.fori_loop(0, NCH * HW // 16, zero_body, 0)

        def m_body(g, mc):
            pv = idxv[pl.ds(g * 16, 16)]
            for j in range(16):
                m = g * 16 + j
                p = pv[j]
                vals = plsc.load_gather(xs, [gat_base + m], mask=m4)
                plsc.addupdate_scatter(buf, [sct_base + p], vals, mask=m4)
            return mc

        lax.fori_loop(0, M // 16, m_body, 0)

        pltpu.sync_copy(buf, out_hbm.at[pl.ds((b * N + cg * NCH) * HW, NCH * HW)])
        return carry

    lax.fori_loop(0, CHUNKS_PER_WORKER, chunk_body, 0)


def kernel(x, spatial_size, location):
    del spatial_size
    loc = location.astype(jnp.int32)
    locy = loc[:, :, 0]
    locx = loc[:, :, 1]
    xt = jnp.transpose(x, (0, 2, 1)).reshape(B, N * M)

    sc = functools.partial(
        pl.kernel,
        out_type=jax.ShapeDtypeStruct((B * N * HW,), jnp.float32),
        mesh=plsc.VectorSubcoreMesh(core_axis_name="c", subcore_axis_name="s"),
        compiler_params=pltpu.CompilerParams(needs_layout_passes=False),
        scratch_types=[
            pltpu.VMEM((M,), jnp.int32),
            pltpu.VMEM((M,), jnp.int32),
            pltpu.VMEM((M,), jnp.int32),
            pltpu.VMEM((NCH * M,), jnp.float32),
            pltpu.VMEM((NCH * HW,), jnp.float32),
        ],
    )(_sc_body)
    out = sc(xt, locy, locx)
    return out.reshape(B, N, H, W)


# speedup vs baseline: 1.9219x; 1.9219x over previous
import functools

import jax
import jax.numpy as jnp
from jax import lax
from jax.experimental import pallas as pl
from jax.experimental.pallas import tpu as pltpu
from jax.experimental.pallas import tpu_sc as plsc

B, M, N = 8, 1024, 256
H, W = 128, 128
HW = H * W
NCH = 4
CHUNKS_PER_WORKER = 16


def _sc_body(xt_hbm, locy_hbm, locx_hbm, out_hbm, yv, xv, idxv, xs, buf):
    c = lax.axis_index("c")
    s = lax.axis_index("s")
    wid = c * 16 + s
    b = wid // 4
    band = wid % 4

    pltpu.sync_copy(locy_hbm.at[b], yv)
    pltpu.sync_copy(locx_hbm.at[b], xv)

    def idx_body(g, carry):
        ys = yv[pl.ds(g * 16, 16)]
        xcs = xv[pl.ds(g * 16, 16)]
        idxv[pl.ds(g * 16, 16)] = ys * W + xcs
        return carry

    lax.fori_loop(0, M // 16, idx_body, 0)

    lanes = lax.iota(jnp.int32, 16)
    m4 = lanes < NCH
    gat_base = jnp.where(m4, lanes * M, 0)
    sct_base = jnp.where(m4, lanes * HW, 0)
    zeros16 = jnp.zeros((16,), jnp.float32)

    def chunk_body(t, carry):
        cg = band * CHUNKS_PER_WORKER + t

        pltpu.sync_copy(xt_hbm.at[b, pl.ds(cg * NCH * M, NCH * M)], xs)

        def zero_body(i, zc):
            for j in range(16):
                buf[pl.ds(i * 256 + j * 16, 16)] = zeros16
            return zc

        lax.fori_loop(0, NCH * HW // 256, zero_body, 0)

        def m_body(g, mc):
            pv = idxv[pl.ds(g * 16, 16)]
            for j in range(16):
                m = g * 16 + j
                p = pv[j]
                vals = plsc.load_gather(xs, [gat_base + m], mask=m4)
                plsc.addupdate_scatter(buf, [sct_base + p], vals, mask=m4)
            return mc

        lax.fori_loop(0, M // 16, m_body, 0)

        pltpu.sync_copy(buf, out_hbm.at[pl.ds((b * N + cg * NCH) * HW, NCH * HW)])
        return carry

    lax.fori_loop(0, CHUNKS_PER_WORKER, chunk_body, 0)


def kernel(x, spatial_size, location):
    del spatial_size
    loc = location.astype(jnp.int32)
    locy = loc[:, :, 0]
    locx = loc[:, :, 1]
    xt = jnp.transpose(x, (0, 2, 1)).reshape(B, N * M)

    sc = functools.partial(
        pl.kernel,
        out_type=jax.ShapeDtypeStruct((B * N * HW,), jnp.float32),
        mesh=plsc.VectorSubcoreMesh(core_axis_name="c", subcore_axis_name="s"),
        compiler_params=pltpu.CompilerParams(needs_layout_passes=False),
        scratch_types=[
            pltpu.VMEM((M,), jnp.int32),
            pltpu.VMEM((M,), jnp.int32),
            pltpu.VMEM((M,), jnp.int32),
            pltpu.VMEM((NCH * M,), jnp.float32),
            pltpu.VMEM((NCH * HW,), jnp.float32),
        ],
    )(_sc_body)
    out = sc(xt, locy, locx)
    return out.reshape(B, N, H, W)


# speedup vs baseline: 2.6438x; 1.3756x over previous
import functools

import jax
import jax.numpy as jnp
from jax import lax
from jax.experimental import pallas as pl
from jax.experimental.pallas import tpu as pltpu
from jax.experimental.pallas import tpu_sc as plsc

B, M, N = 8, 1024, 256
H, W = 128, 128
HW = H * W
NCH = 4
CHUNKS_PER_WORKER = 16


def _sc_body(xt_hbm, locy_hbm, locx_hbm, out_hbm, yv, xv, idxv, xs, buf):
    c = lax.axis_index("c")
    s = lax.axis_index("s")
    wid = c * 16 + s
    b = wid // 4
    band = wid % 4

    pltpu.sync_copy(locy_hbm.at[b], yv)
    pltpu.sync_copy(locx_hbm.at[b], xv)

    def idx_body(g, carry):
        ys = yv[pl.ds(g * 16, 16)]
        xcs = xv[pl.ds(g * 16, 16)]
        idxv[pl.ds(g * 16, 16)] = ys * W + xcs
        return carry

    lax.fori_loop(0, M // 16, idx_body, 0)

    lanes = lax.iota(jnp.int32, 16)
    m4 = lanes < NCH
    gat_base = jnp.where(m4, lanes * M, 0)
    sct_base = jnp.where(m4, lanes * HW, 0)
    zeros16 = jnp.zeros((16,), jnp.float32)

    def chunk_body(t, carry):
        cg = band * CHUNKS_PER_WORKER + t

        pltpu.sync_copy(xt_hbm.at[b, pl.ds(cg * NCH * M, NCH * M)], xs)

        @plsc.parallel_loop(0, NCH * HW // 16, unroll=16)
        def _zero(i):
            buf[pl.ds(i * 16, 16)] = zeros16

        @plsc.parallel_loop(0, M // 16, unroll=2)
        def _accum(g):
            pv = idxv[pl.ds(g * 16, 16)]
            gat0 = gat_base + g * 16
            for j in range(16):
                vals = plsc.load_gather(xs, [gat0 + j], mask=m4)
                plsc.addupdate_scatter(buf, [sct_base + pv[j]], vals, mask=m4)

        pltpu.sync_copy(buf, out_hbm.at[pl.ds((b * N + cg * NCH) * HW, NCH * HW)])
        return carry

    lax.fori_loop(0, CHUNKS_PER_WORKER, chunk_body, 0)


def kernel(x, spatial_size, location):
    del spatial_size
    loc = location.astype(jnp.int32)
    locy = loc[:, :, 0]
    locx = loc[:, :, 1]
    xt = jnp.transpose(x, (0, 2, 1)).reshape(B, N * M)

    sc = functools.partial(
        pl.kernel,
        out_type=jax.ShapeDtypeStruct((B * N * HW,), jnp.float32),
        mesh=plsc.VectorSubcoreMesh(core_axis_name="c", subcore_axis_name="s"),
        compiler_params=pltpu.CompilerParams(needs_layout_passes=False),
        scratch_types=[
            pltpu.VMEM((M,), jnp.int32),
            pltpu.VMEM((M,), jnp.int32),
            pltpu.VMEM((M,), jnp.int32),
            pltpu.VMEM((NCH * M,), jnp.float32),
            pltpu.VMEM((NCH * HW,), jnp.float32),
        ],
    )(_sc_body)
    out = sc(xt, locy, locx)
    return out.reshape(B, N, H, W)


# speedup vs baseline: 3.3386x; 1.2628x over previous
import functools

import jax
import jax.numpy as jnp
from jax import lax
from jax.experimental import pallas as pl
from jax.experimental.pallas import tpu as pltpu
from jax.experimental.pallas import tpu_sc as plsc

B, M, N = 8, 1024, 256
H, W = 128, 128
HW = H * W
NCH = 4
CHUNKS_PER_WORKER = 16


def _sc_body(xt_hbm, locy_hbm, locx_hbm, out_hbm, yv, xv, idxv, xs, buf):
    c = lax.axis_index("c")
    s = lax.axis_index("s")
    wid = c * 16 + s
    b = wid // 4
    band = wid % 4

    pltpu.sync_copy(locy_hbm.at[b], yv)
    pltpu.sync_copy(locx_hbm.at[b], xv)

    def idx_body(g, carry):
        ys = yv[pl.ds(g * 16, 16)]
        xcs = xv[pl.ds(g * 16, 16)]
        idxv[pl.ds(g * 16, 16)] = ys * W + xcs
        return carry

    lax.fori_loop(0, M // 16, idx_body, 0)

    lanes = lax.iota(jnp.int32, 16)
    m4 = lanes < NCH
    gat_base = jnp.where(m4, lanes * M, 0)
    sct_base = jnp.where(m4, lanes * HW, 0)
    zeros16 = jnp.zeros((16,), jnp.float32)

    @plsc.parallel_loop(0, NCH * HW // 16, unroll=16)
    def _zero(i):
        buf[pl.ds(i * 16, 16)] = zeros16

    def chunk_body(t, carry):
        cg = band * CHUNKS_PER_WORKER + t

        pltpu.sync_copy(xt_hbm.at[b, pl.ds(cg * NCH * M, NCH * M)], xs)

        @plsc.parallel_loop(0, M // 16, unroll=2)
        def _accum(g):
            pv = idxv[pl.ds(g * 16, 16)]
            cnt, _ = plsc.scan_count(pv)

            def fast(_):
                for c in range(NCH):
                    vals = xs[pl.ds(c * M + g * 16, 16)]
                    plsc.addupdate_scatter(buf, [pv + c * HW], vals)
                return 0

            def slow(_):
                gat0 = gat_base + g * 16
                for j in range(16):
                    vals = plsc.load_gather(xs, [gat0 + j], mask=m4)
                    plsc.addupdate_scatter(buf, [sct_base + pv[j]], vals,
                                           mask=m4)
                return 0

            lax.cond(jnp.max(cnt) > 1, slow, fast, 0)

        pltpu.sync_copy(buf, out_hbm.at[pl.ds((b * N + cg * NCH) * HW, NCH * HW)])

        @plsc.parallel_loop(0, M // 16, unroll=4)
        def _rezero(g):
            pv = idxv[pl.ds(g * 16, 16)]
            for c in range(NCH):
                plsc.store_scatter(buf, [pv + c * HW], zeros16)

        return carry

    lax.fori_loop(0, CHUNKS_PER_WORKER, chunk_body, 0)


def kernel(x, spatial_size, location):
    del spatial_size
    loc = location.astype(jnp.int32)
    locy = loc[:, :, 0]
    locx = loc[:, :, 1]
    xt = jnp.transpose(x, (0, 2, 1)).reshape(B, N * M)

    sc = functools.partial(
        pl.kernel,
        out_type=jax.ShapeDtypeStruct((B * N * HW,), jnp.float32),
        mesh=plsc.VectorSubcoreMesh(core_axis_name="c", subcore_axis_name="s"),
        compiler_params=pltpu.CompilerParams(needs_layout_passes=False),
        scratch_types=[
            pltpu.VMEM((M,), jnp.int32),
            pltpu.VMEM((M,), jnp.int32),
            pltpu.VMEM((M,), jnp.int32),
            pltpu.VMEM((NCH * M,), jnp.float32),
            pltpu.VMEM((NCH * HW,), jnp.float32),
        ],
    )(_sc_body)
    out = sc(xt, locy, locx)
    return out.reshape(B, N, H, W)
